# baseline (speedup 1.0000x reference)
import jax
import jax.numpy as jnp
from jax import lax
from jax.experimental import pallas as pl
from jax.experimental.pallas import tpu as pltpu
from jax.experimental.pallas import tpu_sc as plsc

_NC = 2
_NS = 16
_NW = _NC * _NS
_E = 320000
_D = 128
_N_NODE = 10000
_CB = 80
_NG = _CB // 16
_NCHUNK = (_E // _NW) // _CB

_LOG2E = 1.4426950408889634
_LN2 = 0.6931471805599453


def _exp_neg(x):
    x = jnp.maximum(x, -80.0)
    k = (x * _LOG2E).astype(jnp.int32)
    r = x - k.astype(jnp.float32) * _LN2
    er = 1.0 + r * (1.0 + r * (1.0 / 2.0) * (1.0 + r * (1.0 / 3.0) * (
        1.0 + r * (1.0 / 4.0) * (1.0 + r * (1.0 / 5.0) * (
            1.0 + r * (1.0 / 6.0) * (1.0 + r * (1.0 / 7.0) * (
                1.0 + r * (1.0 / 8.0) * (1.0 + r * (1.0 / 9.0)))))))))
    scale = lax.bitcast_convert_type((k + 127) << 23, jnp.float32)
    return scale * er


def _expm1_01(x):
    return x * (1.0 + x * (1.0 / 2.0) * (1.0 + x * (1.0 / 3.0) * (
        1.0 + x * (1.0 / 4.0) * (1.0 + x * (1.0 / 5.0) * (
            1.0 + x * (1.0 / 6.0) * (1.0 + x * (1.0 / 7.0) * (
                1.0 + x * (1.0 / 8.0) * (1.0 + x * (1.0 / 9.0) * (
                    1.0 + x * (1.0 / 10.0))))))))))


def _recip(d):
    y = 1.0 / d
    y = y * (2.0 - d * y)
    return y * (2.0 - d * y)


def _sigmoid(a):
    t = _exp_neg(-jnp.abs(a))
    inv = _recip(1.0 + t)
    return jnp.where(a >= 0, inv, t * inv)


def _sc_body(u_hbm, i_hbm, src_hbm, dst_hbm, ge_hbm, sz_hbm, w_hbm, b_hbm,
             mu_hbm, disp_hbm, pi_hbm,
             idx_v0, idx_v1, u0, u1, i0, i1, o0, o1,
             ge_v, sz_v, w_v, b_v, pre_m, pre_d, pre_p,
             si0, si1, su0, su1, sv0, sv1):
    wid = lax.axis_index("s") * _NC + lax.axis_index("c")
    epw = _E // _NW
    base = wid * epw

    bufs = ((idx_v0, u0, i0, o0, si0, su0, sv0),
            (idx_v1, u1, i1, o1, si1, su1, sv1))

    pltpu.sync_copy(ge_hbm, ge_v)
    pltpu.sync_copy(sz_hbm, sz_v)
    pltpu.sync_copy(w_hbm, w_v)
    pltpu.sync_copy(b_hbm, b_v)

    lanes = lax.iota(jnp.int32, 16)
    zero16 = jnp.zeros((16,), jnp.int32)
    one16 = zero16 + 1
    two16 = zero16 + 2
    three16 = zero16 + 3
    bm = plsc.load_gather(b_v, [one16])
    bd = plsc.load_gather(b_v, [two16])
    bp = plsc.load_gather(b_v, [three16])

    def start_idx(x, b):
        idx_v, _, _, _, s_idx, _, _ = bufs[b]
        off = base + x * _CB
        pltpu.async_copy(src_hbm.at[pl.ds(off, _CB)], idx_v.at[0], s_idx)
        pltpu.async_copy(dst_hbm.at[pl.ds(off, _CB)], idx_v.at[1], s_idx)

    def start_gather(b):
        idx_v, u_r, i_r, _, s_idx, s_u, s_i = bufs[b]
        pltpu.make_async_copy(src_hbm.at[pl.ds(0, _CB)], idx_v.at[0], s_idx).wait()
        pltpu.make_async_copy(dst_hbm.at[pl.ds(0, _CB)], idx_v.at[1], s_idx).wait()
        pltpu.async_copy(u_hbm.at[idx_v.at[0]], u_r, s_u)
        pltpu.async_copy(i_hbm.at[idx_v.at[1]], i_r, s_i)

    def compute(x, b):
        idx_v, u_r, i_r, o_v, _, s_u, s_i = bufs[b]
        pltpu.make_async_copy(u_hbm.at[idx_v.at[0]], u_r, s_u).wait()
        pltpu.make_async_copy(i_hbm.at[idx_v.at[1]], i_r, s_i).wait()

        rows = [lanes + g * 16 for g in range(_NG)]

        zacc = tuple((jnp.zeros((16,), jnp.float32),) * 3 for _ in range(_NG))

        @plsc.parallel_loop(0, _D, step=4, carry=zacc)
        def accs(jb, accs):
            accs = list(accs)
            for dj in range(4):
                j = jb + dj
                colv = (lanes + j) & 127
                wm = plsc.load_gather(w_v, [zero16, colv])
                wd = plsc.load_gather(w_v, [one16, colv])
                wp = plsc.load_gather(w_v, [two16, colv])
                out = []
                for g in range(_NG):
                    am, ad, ap = accs[g]
                    u = plsc.load_gather(u_r, [rows[g], colv])
                    iv = plsc.load_gather(i_r, [rows[g], colv])
                    pr = u * iv
                    out.append((am + pr * wm, ad + pr * wd, ap + pr * wp))
                accs = out
            return tuple(accs)

        for g in range(_NG):
            am, ad, ap = accs[g]
            src16 = idx_v[0, pl.ds(g * 16, 16)]
            dst16 = idx_v[1, pl.ds(g * 16, 16)]
            ge = plsc.load_gather(ge_v, [dst16])
            sz = plsc.load_gather(sz_v, [src16])
            mu_s = _sigmoid(am + bm)
            pi_s = _sigmoid(ap + bp)
            z = ge * (ad + bd)
            t = _exp_neg(-jnp.abs(z))
            uu = t * _recip(2.0 + t)
            u2 = uu * uu
            l1p = 2.0 * uu * (1.0 + u2 * (1.0 / 3.0 + u2 * (0.2 + u2 * (1.0 / 7.0))))
            sp = jnp.maximum(z, 0.0) + l1p
            disp = jnp.clip(sp, 1e-4, 1e4)
            mu = sz * jnp.clip(_expm1_01(ge * mu_s), 1e-5, 1e6)
            o_v[0, pl.ds(g * 16, 16)] = mu
            o_v[1, pl.ds(g * 16, 16)] = disp
            o_v[2, pl.ds(g * 16, 16)] = pi_s

        off = base + x * _CB
        pltpu.sync_copy(o_v.at[0], mu_hbm.at[pl.ds(off, _CB)])
        pltpu.sync_copy(o_v.at[1], disp_hbm.at[pl.ds(off, _CB)])
        pltpu.sync_copy(o_v.at[2], pi_hbm.at[pl.ds(off, _CB)])

    start_idx(0, 0)
    start_gather(0)

    def two(k, carry):
        x0 = 2 * k
        start_idx(x0 + 1, 1)
        start_gather(1)
        compute(x0, 0)
        start_idx(x0 + 2, 0)
        start_gather(0)
        compute(x0 + 1, 1)
        return carry

    lax.fori_loop(0, (_NCHUNK - 1) // 2, two, 0)
    compute(_NCHUNK - 1, 0)


@jax.jit
def _run(ufeats, ifeats, src, dst, ge, sz, w, b):
    f = pl.kernel(
        _sc_body,
        out_type=(jax.ShapeDtypeStruct((_E,), jnp.float32),) * 3,
        mesh=plsc.VectorSubcoreMesh(core_axis_name="c", subcore_axis_name="s"),
        compiler_params=pltpu.CompilerParams(needs_layout_passes=False),
        scratch_types=[
            pltpu.VMEM((2, _CB), jnp.int32),
            pltpu.VMEM((2, _CB), jnp.int32),
            pltpu.VMEM((_CB, _D), jnp.float32),
            pltpu.VMEM((_CB, _D), jnp.float32),
            pltpu.VMEM((_CB, _D), jnp.float32),
            pltpu.VMEM((_CB, _D), jnp.float32),
            pltpu.VMEM((3, _CB), jnp.float32),
            pltpu.VMEM((3, _CB), jnp.float32),
            pltpu.VMEM((_N_NODE,), jnp.float32),
            pltpu.VMEM((_N_NODE,), jnp.float32),
            pltpu.VMEM((3, _D), jnp.float32),
            pltpu.VMEM((16,), jnp.float32),
            pltpu.VMEM((_CB,), jnp.float32),
            pltpu.VMEM((_CB,), jnp.float32),
            pltpu.VMEM((_CB,), jnp.float32),
            pltpu.SemaphoreType.DMA,
            pltpu.SemaphoreType.DMA,
            pltpu.SemaphoreType.DMA,
            pltpu.SemaphoreType.DMA,
            pltpu.SemaphoreType.DMA,
            pltpu.SemaphoreType.DMA,
        ],
    )
    return f(ufeats, ifeats, src, dst, ge, sz, w, b)


def kernel(ufeats, ifeats, edge_index, ge_factor, sz_factor,
           W_mean, b_mean, W_disp, b_disp, W_pi, b_pi):
    src = edge_index[0].astype(jnp.int32)
    dst = edge_index[1].astype(jnp.int32)
    ge = ge_factor.reshape(-1)
    sz = sz_factor.reshape(-1)
    w = jnp.stack([W_mean[:, 0], W_disp[:, 0], W_pi[:, 0]])
    b = jnp.concatenate(
        [jnp.zeros((1,), jnp.float32), b_mean, b_disp, b_pi,
         jnp.zeros((12,), jnp.float32)]).astype(jnp.float32)
    mu, disp, pi = _run(ufeats, ifeats, src, dst, ge, sz, w, b)
    return (mu.reshape(_E, 1), disp.reshape(_E, 1), pi.reshape(_E, 1))


# speedup vs baseline: 1.0942x; 1.0942x over previous
import jax
import jax.numpy as jnp
from jax import lax
from jax.experimental import pallas as pl
from jax.experimental.pallas import tpu as pltpu
from jax.experimental.pallas import tpu_sc as plsc

_NC = 2
_NS = 16
_NW = _NC * _NS
_E = 320000
_D = 128
_N_NODE = 10000
_CB = 80
_NG = _CB // 16
_NCHUNK = (_E // _NW) // _CB

_LOG2E = 1.4426950408889634
_LN2 = 0.6931471805599453


def _exp_neg(x):
    x = jnp.maximum(x, -80.0)
    k = (x * _LOG2E).astype(jnp.int32)
    r = x - k.astype(jnp.float32) * _LN2
    er = 1.0 + r * (1.0 + r * (1.0 / 2.0) * (1.0 + r * (1.0 / 3.0) * (
        1.0 + r * (1.0 / 4.0) * (1.0 + r * (1.0 / 5.0) * (
            1.0 + r * (1.0 / 6.0) * (1.0 + r * (1.0 / 7.0) * (
                1.0 + r * (1.0 / 8.0) * (1.0 + r * (1.0 / 9.0)))))))))
    scale = lax.bitcast_convert_type((k + 127) << 23, jnp.float32)
    return scale * er


def _expm1_01(x):
    return x * (1.0 + x * (1.0 / 2.0) * (1.0 + x * (1.0 / 3.0) * (
        1.0 + x * (1.0 / 4.0) * (1.0 + x * (1.0 / 5.0) * (
            1.0 + x * (1.0 / 6.0) * (1.0 + x * (1.0 / 7.0) * (
                1.0 + x * (1.0 / 8.0) * (1.0 + x * (1.0 / 9.0) * (
                    1.0 + x * (1.0 / 10.0))))))))))


def _recip(d):
    y = 1.0 / d
    y = y * (2.0 - d * y)
    return y * (2.0 - d * y)


def _sigmoid(a):
    t = _exp_neg(-jnp.abs(a))
    inv = _recip(1.0 + t)
    return jnp.where(a >= 0, inv, t * inv)


def _sc_body(u_hbm, i_hbm, src_hbm, dst_hbm, ge_hbm, sz_hbm, w_hbm, b_hbm,
             mu_hbm, disp_hbm, pi_hbm,
             idx_v0, idx_v1, u0, u1, i0, i1, o0, o1,
             ge_v, sz_v, w_v, b_v,
             si0, si1, su0, su1, sv0, sv1, so0, so1):
    wid = lax.axis_index("s") * _NC + lax.axis_index("c")
    epw = _E // _NW
    base = wid * epw

    bufs = ((idx_v0, u0, i0, o0, si0, su0, sv0, so0),
            (idx_v1, u1, i1, o1, si1, su1, sv1, so1))

    pltpu.sync_copy(ge_hbm, ge_v)
    pltpu.sync_copy(sz_hbm, sz_v)
    pltpu.sync_copy(w_hbm, w_v)
    pltpu.sync_copy(b_hbm, b_v)

    lanes = lax.iota(jnp.int32, 16)
    zero16 = jnp.zeros((16,), jnp.int32)
    one16 = zero16 + 1
    two16 = zero16 + 2
    three16 = zero16 + 3
    bm = plsc.load_gather(b_v, [one16])
    bd = plsc.load_gather(b_v, [two16])
    bp = plsc.load_gather(b_v, [three16])

    def start_idx(x, b):
        idx_v, _, _, _, s_idx, _, _, _ = bufs[b]
        off = base + jnp.minimum(x, _NCHUNK - 1) * _CB
        pltpu.async_copy(src_hbm.at[pl.ds(off, _CB)], idx_v.at[0], s_idx)
        pltpu.async_copy(dst_hbm.at[pl.ds(off, _CB)], idx_v.at[1], s_idx)

    def start_gather(b):
        idx_v, u_r, i_r, _, s_idx, s_u, s_i, _ = bufs[b]
        pltpu.make_async_copy(src_hbm.at[pl.ds(0, _CB)], idx_v.at[0], s_idx).wait()
        pltpu.make_async_copy(dst_hbm.at[pl.ds(0, _CB)], idx_v.at[1], s_idx).wait()
        pltpu.async_copy(u_hbm.at[idx_v.at[0]], u_r, s_u)
        pltpu.async_copy(i_hbm.at[idx_v.at[1]], i_r, s_i)

    def wait_store(b):
        _, _, _, o_v, _, _, _, s_o = bufs[b]
        pltpu.make_async_copy(o_v.at[0], mu_hbm.at[pl.ds(0, _CB)], s_o).wait()
        pltpu.make_async_copy(o_v.at[1], disp_hbm.at[pl.ds(0, _CB)], s_o).wait()
        pltpu.make_async_copy(o_v.at[2], pi_hbm.at[pl.ds(0, _CB)], s_o).wait()

    def compute(x, b):
        idx_v, u_r, i_r, o_v, _, s_u, s_i, s_o = bufs[b]
        pltpu.make_async_copy(u_hbm.at[idx_v.at[0]], u_r, s_u).wait()
        pltpu.make_async_copy(i_hbm.at[idx_v.at[1]], i_r, s_i).wait()

        geL = [plsc.load_gather(ge_v, [idx_v[1, pl.ds(g * 16, 16)]])
               for g in range(_NG)]
        szL = [plsc.load_gather(sz_v, [idx_v[0, pl.ds(g * 16, 16)]])
               for g in range(_NG)]
        start_idx(x + 2, b)

        rows = [lanes + g * 16 for g in range(_NG)]

        def jstep(jb, accs):
            accs = list(accs)
            for dj in range(4):
                j = jb * 4 + dj
                colv = (lanes + j) & 127
                wm = plsc.load_gather(w_v, [zero16, colv])
                wd = plsc.load_gather(w_v, [one16, colv])
                wp = plsc.load_gather(w_v, [two16, colv])
                out = []
                for g in range(_NG):
                    am, ad, ap = accs[g]
                    u = plsc.load_gather(u_r, [rows[g], colv])
                    iv = plsc.load_gather(i_r, [rows[g], colv])
                    pr = u * iv
                    out.append((am + pr * wm, ad + pr * wd, ap + pr * wp))
                accs = out
            return tuple(accs)

        zacc = tuple((jnp.zeros((16,), jnp.float32),) * 3 for _ in range(_NG))
        accs = lax.fori_loop(0, _D // 4, jstep, zacc)

        @pl.when(x >= 2)
        def _():
            wait_store(b)

        for g in range(_NG):
            am, ad, ap = accs[g]
            ge = geL[g]
            sz = szL[g]
            mu_s = _sigmoid(am + bm)
            pi_s = _sigmoid(ap + bp)
            z = ge * (ad + bd)
            t = _exp_neg(-jnp.abs(z))
            uu = t * _recip(2.0 + t)
            u2 = uu * uu
            l1p = 2.0 * uu * (1.0 + u2 * (1.0 / 3.0 + u2 * (0.2 + u2 * (1.0 / 7.0))))
            sp = jnp.maximum(z, 0.0) + l1p
            disp = jnp.clip(sp, 1e-4, 1e4)
            mu = sz * jnp.clip(_expm1_01(ge * mu_s), 1e-5, 1e6)
            o_v[0, pl.ds(g * 16, 16)] = mu
            o_v[1, pl.ds(g * 16, 16)] = disp
            o_v[2, pl.ds(g * 16, 16)] = pi_s

        off = base + x * _CB
        pltpu.async_copy(o_v.at[0], mu_hbm.at[pl.ds(off, _CB)], s_o)
        pltpu.async_copy(o_v.at[1], disp_hbm.at[pl.ds(off, _CB)], s_o)
        pltpu.async_copy(o_v.at[2], pi_hbm.at[pl.ds(off, _CB)], s_o)

    start_idx(0, 0)
    start_gather(0)
    start_idx(1, 1)

    def two(k, carry):
        x0 = 2 * k
        start_gather(1)
        compute(x0, 0)
        start_gather(0)
        compute(x0 + 1, 1)
        return carry

    lax.fori_loop(0, (_NCHUNK - 1) // 2, two, 0)
    compute(_NCHUNK - 1, 0)
    wait_store(0)
    wait_store(1)


@jax.jit
def _run(ufeats, ifeats, src, dst, ge, sz, w, b):
    f = pl.kernel(
        _sc_body,
        out_type=(jax.ShapeDtypeStruct((_E,), jnp.float32),) * 3,
        mesh=plsc.VectorSubcoreMesh(core_axis_name="c", subcore_axis_name="s"),
        compiler_params=pltpu.CompilerParams(needs_layout_passes=False),
        scratch_types=[
            pltpu.VMEM((2, _CB), jnp.int32),
            pltpu.VMEM((2, _CB), jnp.int32),
            pltpu.VMEM((_CB, _D), jnp.float32),
            pltpu.VMEM((_CB, _D), jnp.float32),
            pltpu.VMEM((_CB, _D), jnp.float32),
            pltpu.VMEM((_CB, _D), jnp.float32),
            pltpu.VMEM((3, _CB), jnp.float32),
            pltpu.VMEM((3, _CB), jnp.float32),
            pltpu.VMEM((_N_NODE,), jnp.float32),
            pltpu.VMEM((_N_NODE,), jnp.float32),
            pltpu.VMEM((3, _D), jnp.float32),
            pltpu.VMEM((16,), jnp.float32),
            pltpu.SemaphoreType.DMA,
            pltpu.SemaphoreType.DMA,
            pltpu.SemaphoreType.DMA,
            pltpu.SemaphoreType.DMA,
            pltpu.SemaphoreType.DMA,
            pltpu.SemaphoreType.DMA,
            pltpu.SemaphoreType.DMA,
            pltpu.SemaphoreType.DMA,
        ],
    )
    return f(ufeats, ifeats, src, dst, ge, sz, w, b)


def kernel(ufeats, ifeats, edge_index, ge_factor, sz_factor,
           W_mean, b_mean, W_disp, b_disp, W_pi, b_pi):
    src = edge_index[0].astype(jnp.int32)
    dst = edge_index[1].astype(jnp.int32)
    ge = ge_factor.reshape(-1)
    sz = sz_factor.reshape(-1)
    w = jnp.stack([W_mean[:, 0], W_disp[:, 0], W_pi[:, 0]])
    b = jnp.concatenate(
        [jnp.zeros((1,), jnp.float32), b_mean, b_disp, b_pi,
         jnp.zeros((12,), jnp.float32)]).astype(jnp.float32)
    mu, disp, pi = _run(ufeats, ifeats, src, dst, ge, sz, w, b)
    return (mu.reshape(_E, 1), disp.reshape(_E, 1), pi.reshape(_E, 1))


# speedup vs baseline: 1.6023x; 1.4644x over previous
import jax
import jax.numpy as jnp
from jax import lax
from jax.experimental import pallas as pl
from jax.experimental.pallas import tpu as pltpu
from jax.experimental.pallas import tpu_sc as plsc

_NC = 2
_NS = 16
_NW = _NC * _NS
_E = 320000
_D = 128
_N_NODE = 10000
_CB = 80
_NG = _CB // 16
_NCHUNK = (_E // _NW) // _CB

_LOG2E = 1.4426950408889634
_LN2 = 0.6931471805599453


def _exp_neg(x):
    x = jnp.maximum(x, -80.0)
    k = (x * _LOG2E).astype(jnp.int32)
    r = x - k.astype(jnp.float32) * _LN2
    er = 1.0 + r * (1.0 + r * (1.0 / 2.0) * (1.0 + r * (1.0 / 3.0) * (
        1.0 + r * (1.0 / 4.0) * (1.0 + r * (1.0 / 5.0) * (
            1.0 + r * (1.0 / 6.0) * (1.0 + r * (1.0 / 7.0) * (
                1.0 + r * (1.0 / 8.0) * (1.0 + r * (1.0 / 9.0)))))))))
    scale = lax.bitcast_convert_type((k + 127) << 23, jnp.float32)
    return scale * er


def _expm1_01(x):
    return x * (1.0 + x * (1.0 / 2.0) * (1.0 + x * (1.0 / 3.0) * (
        1.0 + x * (1.0 / 4.0) * (1.0 + x * (1.0 / 5.0) * (
            1.0 + x * (1.0 / 6.0) * (1.0 + x * (1.0 / 7.0) * (
                1.0 + x * (1.0 / 8.0) * (1.0 + x * (1.0 / 9.0) * (
                    1.0 + x * (1.0 / 10.0))))))))))


def _recip(d):
    y = 1.0 / d
    y = y * (2.0 - d * y)
    return y * (2.0 - d * y)


def _sigmoid(a):
    t = _exp_neg(-jnp.abs(a))
    inv = _recip(1.0 + t)
    return jnp.where(a >= 0, inv, t * inv)


def _sc_body(u_hbm, i_hbm, src_hbm, dst_hbm, ge_hbm, sz_hbm, w_hbm, b_hbm,
             mu_hbm, disp_hbm, pi_hbm,
             idx_v0, idx_v1, u0, u1, i0, i1, o0, o1,
             ge_v, sz_v, w_v, b_v, wrot_m, wrot_d, wrot_p,
             si0, si1, su0, su1, sv0, sv1, so0, so1):
    wid = lax.axis_index("s") * _NC + lax.axis_index("c")
    epw = _E // _NW
    base = wid * epw

    bufs = ((idx_v0, u0, i0, o0, si0, su0, sv0, so0),
            (idx_v1, u1, i1, o1, si1, su1, sv1, so1))

    pltpu.sync_copy(ge_hbm, ge_v)
    pltpu.sync_copy(sz_hbm, sz_v)
    pltpu.sync_copy(w_hbm, w_v)
    pltpu.sync_copy(b_hbm, b_v)

    lanes = lax.iota(jnp.int32, 16)
    zero16 = jnp.zeros((16,), jnp.int32)
    one16 = zero16 + 1
    two16 = zero16 + 2
    three16 = zero16 + 3
    bm = plsc.load_gather(b_v, [one16])
    bd = plsc.load_gather(b_v, [two16])
    bp = plsc.load_gather(b_v, [three16])

    def winit(j, carry):
        colv = (lanes + j) & 127
        wrot_m[j, pl.ds(0, 16)] = plsc.load_gather(w_v, [zero16, colv])
        wrot_d[j, pl.ds(0, 16)] = plsc.load_gather(w_v, [one16, colv])
        wrot_p[j, pl.ds(0, 16)] = plsc.load_gather(w_v, [two16, colv])
        return carry

    lax.fori_loop(0, _D, winit, 0)

    def start_idx(x, b):
        idx_v, _, _, _, s_idx, _, _, _ = bufs[b]
        off = base + jnp.minimum(x, _NCHUNK - 1) * _CB
        pltpu.async_copy(src_hbm.at[pl.ds(off, _CB)], idx_v.at[0], s_idx)
        pltpu.async_copy(dst_hbm.at[pl.ds(off, _CB)], idx_v.at[1], s_idx)

    def start_gather(b):
        idx_v, u_r, i_r, _, s_idx, s_u, s_i, _ = bufs[b]
        pltpu.make_async_copy(src_hbm.at[pl.ds(0, _CB)], idx_v.at[0], s_idx).wait()
        pltpu.make_async_copy(dst_hbm.at[pl.ds(0, _CB)], idx_v.at[1], s_idx).wait()
        pltpu.async_copy(u_hbm.at[idx_v.at[0]], u_r, s_u)
        pltpu.async_copy(i_hbm.at[idx_v.at[1]], i_r, s_i)

    def wait_store(b):
        _, _, _, o_v, _, _, _, s_o = bufs[b]
        pltpu.make_async_copy(o_v.at[0], mu_hbm.at[pl.ds(0, _CB)], s_o).wait()
        pltpu.make_async_copy(o_v.at[1], disp_hbm.at[pl.ds(0, _CB)], s_o).wait()
        pltpu.make_async_copy(o_v.at[2], pi_hbm.at[pl.ds(0, _CB)], s_o).wait()

    def compute(x, b):
        idx_v, u_r, i_r, o_v, _, s_u, s_i, s_o = bufs[b]
        pltpu.make_async_copy(u_hbm.at[idx_v.at[0]], u_r, s_u).wait()
        pltpu.make_async_copy(i_hbm.at[idx_v.at[1]], i_r, s_i).wait()

        geL = [plsc.load_gather(ge_v, [idx_v[1, pl.ds(g * 16, 16)]])
               for g in range(_NG)]
        szL = [plsc.load_gather(sz_v, [idx_v[0, pl.ds(g * 16, 16)]])
               for g in range(_NG)]
        start_idx(x + 2, b)

        rows = [lanes + g * 16 for g in range(_NG)]

        def jstep(jb, accs):
            accs = list(accs)
            for dj in range(16):
                j = jb * 16 + dj
                colv = (lanes + j) & 127
                wm = wrot_m[j, pl.ds(0, 16)]
                wd = wrot_d[j, pl.ds(0, 16)]
                wp = wrot_p[j, pl.ds(0, 16)]
                out = []
                for g in range(_NG):
                    am, ad, ap = accs[g]
                    u = plsc.load_gather(u_r, [rows[g], colv])
                    iv = plsc.load_gather(i_r, [rows[g], colv])
                    pr = u * iv
                    out.append((am + pr * wm, ad + pr * wd, ap + pr * wp))
                accs = out
            return tuple(accs)

        zacc = tuple((jnp.zeros((16,), jnp.float32),) * 3 for _ in range(_NG))
        accs = lax.fori_loop(0, _D // 16, jstep, zacc)

        @pl.when(x >= 2)
        def _():
            wait_store(b)

        for g in range(_NG):
            am, ad, ap = accs[g]
            ge = geL[g]
            sz = szL[g]
            mu_s = _sigmoid(am + bm)
            pi_s = _sigmoid(ap + bp)
            z = ge * (ad + bd)
            t = _exp_neg(-jnp.abs(z))
            uu = t * _recip(2.0 + t)
            u2 = uu * uu
            l1p = 2.0 * uu * (1.0 + u2 * (1.0 / 3.0 + u2 * (0.2 + u2 * (1.0 / 7.0))))
            sp = jnp.maximum(z, 0.0) + l1p
            disp = jnp.clip(sp, 1e-4, 1e4)
            mu = sz * jnp.clip(_expm1_01(ge * mu_s), 1e-5, 1e6)
            o_v[0, pl.ds(g * 16, 16)] = mu
            o_v[1, pl.ds(g * 16, 16)] = disp
            o_v[2, pl.ds(g * 16, 16)] = pi_s

        off = base + x * _CB
        pltpu.async_copy(o_v.at[0], mu_hbm.at[pl.ds(off, _CB)], s_o)
        pltpu.async_copy(o_v.at[1], disp_hbm.at[pl.ds(off, _CB)], s_o)
        pltpu.async_copy(o_v.at[2], pi_hbm.at[pl.ds(off, _CB)], s_o)

    start_idx(0, 0)
    start_gather(0)
    start_idx(1, 1)

    def two(k, carry):
        x0 = 2 * k
        start_gather(1)
        compute(x0, 0)
        start_gather(0)
        compute(x0 + 1, 1)
        return carry

    lax.fori_loop(0, (_NCHUNK - 1) // 2, two, 0)
    compute(_NCHUNK - 1, 0)
    wait_store(0)
    wait_store(1)


@jax.jit
def _run(ufeats, ifeats, src, dst, ge, sz, w, b):
    f = pl.kernel(
        _sc_body,
        out_type=(jax.ShapeDtypeStruct((_E,), jnp.float32),) * 3,
        mesh=plsc.VectorSubcoreMesh(core_axis_name="c", subcore_axis_name="s"),
        compiler_params=pltpu.CompilerParams(needs_layout_passes=False),
        scratch_types=[
            pltpu.VMEM((2, _CB), jnp.int32),
            pltpu.VMEM((2, _CB), jnp.int32),
            pltpu.VMEM((_CB, _D), jnp.float32),
            pltpu.VMEM((_CB, _D), jnp.float32),
            pltpu.VMEM((_CB, _D), jnp.float32),
            pltpu.VMEM((_CB, _D), jnp.float32),
            pltpu.VMEM((3, _CB), jnp.float32),
            pltpu.VMEM((3, _CB), jnp.float32),
            pltpu.VMEM((_N_NODE,), jnp.float32),
            pltpu.VMEM((_N_NODE,), jnp.float32),
            pltpu.VMEM((3, _D), jnp.float32),
            pltpu.VMEM((16,), jnp.float32),
            pltpu.VMEM((_D, 16), jnp.float32),
            pltpu.VMEM((_D, 16), jnp.float32),
            pltpu.VMEM((_D, 16), jnp.float32),
            pltpu.SemaphoreType.DMA,
            pltpu.SemaphoreType.DMA,
            pltpu.SemaphoreType.DMA,
            pltpu.SemaphoreType.DMA,
            pltpu.SemaphoreType.DMA,
            pltpu.SemaphoreType.DMA,
            pltpu.SemaphoreType.DMA,
            pltpu.SemaphoreType.DMA,
        ],
    )
    return f(ufeats, ifeats, src, dst, ge, sz, w, b)


def kernel(ufeats, ifeats, edge_index, ge_factor, sz_factor,
           W_mean, b_mean, W_disp, b_disp, W_pi, b_pi):
    src = edge_index[0].astype(jnp.int32)
    dst = edge_index[1].astype(jnp.int32)
    ge = ge_factor.reshape(-1)
    sz = sz_factor.reshape(-1)
    w = jnp.stack([W_mean[:, 0], W_disp[:, 0], W_pi[:, 0]])
    b = jnp.concatenate(
        [jnp.zeros((1,), jnp.float32), b_mean, b_disp, b_pi,
         jnp.zeros((12,), jnp.float32)]).astype(jnp.float32)
    mu, disp, pi = _run(ufeats, ifeats, src, dst, ge, sz, w, b)
    return (mu.reshape(_E, 1), disp.reshape(_E, 1), pi.reshape(_E, 1))
